# Initial kernel scaffold; baseline (speedup 1.0000x reference)
import jax
import jax.numpy as jnp
from jax.experimental import pallas as pl


def kernel(x, edge_index, edge_attr, params):
    raise NotImplementedError("write your pallas kernel here")


# speedup vs baseline: 4.9541x; 4.9541x over previous
import functools

import jax
import jax.numpy as jnp
from jax import lax
from jax.experimental import pallas as pl
from jax.experimental.pallas import tpu as pltpu
from jax.experimental.pallas import tpu_sc as plsc

N = 10000
E = 320000
NC = 2
NS = 16
NW = NC * NS
K = 80
CPT = E // (NW * K)
NP = 10240
RPT = NP // NS

_MESH = plsc.VectorSubcoreMesh(core_axis_name="c", subcore_axis_name="s")


def _zero_vmem(ref, rows, width):
    def body(t, carry):
        i = t // (width // 16)
        j = t % (width // 16)
        ref[i, pl.ds(j * 16, 16)] = jnp.zeros((16,), jnp.float32)
        return carry
    lax.fori_loop(0, rows * (width // 16), body, None)


def _make_seg_sum(W):
    @functools.partial(
        pl.kernel,
        out_type=jax.ShapeDtypeStruct((NC, NP, W), jnp.float32),
        mesh=_MESH,
        scratch_types=[
            pltpu.VMEM((CPT, K), jnp.int32),
            pltpu.VMEM((CPT, K), jnp.int32),
            pltpu.VMEM((K, W), jnp.float32),
            pltpu.VMEM_SHARED((NP, W), jnp.float32),
            pltpu.SemaphoreType.DMA,
        ],
    )
    def seg_sum(h_hbm, src_hbm, dst_hbm, out_hbm,
                src_v, dst_v, rows_v, acc_sh, sem):
        c = lax.axis_index("c")
        s = lax.axis_index("s")
        wid = s * NC + c

        _zero_vmem(rows_v, K, W)
        for b in range(RPT // K):
            pltpu.sync_copy(rows_v, acc_sh.at[pl.ds(s * RPT + b * K, K)])
        pltpu.sync_copy(src_hbm.at[wid], src_v)
        pltpu.sync_copy(dst_hbm.at[wid], dst_v)
        plsc.subcore_barrier()

        def chunk(j, carry):
            pltpu.async_copy(h_hbm.at[src_v.at[j]], rows_v, sem).wait()
            pltpu.sync_copy(rows_v, acc_sh.at[dst_v.at[j]], add=True)
            return carry
        lax.fori_loop(0, CPT, chunk, None)

        plsc.subcore_barrier()
        for b in range(RPT // K):
            off = s * RPT + b * K
            pltpu.sync_copy(acc_sh.at[pl.ds(off, K)], rows_v)
            pltpu.sync_copy(rows_v, out_hbm.at[c].at[pl.ds(off, K)])

    return seg_sum


_SEG_SUM_128 = _make_seg_sum(128)


_R = 1000


def _finalize_prepro(pre):
    def body(pre_ref, ean_ref, inv_ref):
        p = pre_ref[0] + pre_ref[1]
        ean_ref[...] = p[:, 0:16]
        deg = p[:, 16:17]
        inv_ref[...] = jnp.broadcast_to(1.0 / jnp.maximum(deg, 1.0), (_R, 16))

    return pl.pallas_call(
        body,
        grid=(N // _R,),
        in_specs=[pl.BlockSpec((NC, _R, 128), lambda r: (0, r, 0))],
        out_specs=[
            pl.BlockSpec((_R, 16), lambda r: (r, 0)),
            pl.BlockSpec((_R, 16), lambda r: (r, 0)),
        ],
        out_shape=[
            jax.ShapeDtypeStruct((N, 16), jnp.float32),
            jax.ShapeDtypeStruct((N, 16), jnp.float32),
        ],
    )(pre)


def _dense_layer(h_blocks, g_parts, ean, inv, w_self, w_nbr, w_edge, bias,
                 in_widths, out_widths, relu):
    nb = len(h_blocks)
    fo = sum(out_widths)

    def body(*refs):
        h_refs = refs[:nb]
        g_refs = refs[nb:2 * nb]
        ean_ref, inv_ref, ws_ref, wn_ref, we_ref, b_ref = refs[2 * nb:2 * nb + 6]
        out_refs = refs[2 * nb + 6:]

        acc = b_ref[...].astype(jnp.float32)
        nbr = jnp.zeros((_R, fo), jnp.float32)
        off = 0
        for bi in range(nb):
            w = in_widths[bi]
            acc = acc + jnp.dot(h_refs[bi][...], ws_ref[off:off + w, :],
                                preferred_element_type=jnp.float32)
            g = g_refs[bi][0] + g_refs[bi][1]
            nbr = nbr + jnp.dot(g, wn_ref[off:off + w, :],
                                preferred_element_type=jnp.float32)
            off += w
        nbr = nbr + jnp.dot(ean_ref[...], we_ref[...],
                            preferred_element_type=jnp.float32)
        y = acc + nbr * inv_ref[...][:, 0:1]
        if relu:
            y = jnp.maximum(y, 0.0)
        off = 0
        for oi, w in enumerate(out_widths):
            out_refs[oi][...] = y[:, off:off + w]
            off += w

    fi = sum(in_widths)
    in_specs = (
        [pl.BlockSpec((_R, w), lambda r: (r, 0)) for w in in_widths]
        + [pl.BlockSpec((NC, _R, w), lambda r: (0, r, 0)) for w in in_widths]
        + [
            pl.BlockSpec((_R, 16), lambda r: (r, 0)),
            pl.BlockSpec((_R, 16), lambda r: (r, 0)),
            pl.BlockSpec((fi, fo), lambda r: (0, 0)),
            pl.BlockSpec((fi, fo), lambda r: (0, 0)),
            pl.BlockSpec((16, fo), lambda r: (0, 0)),
            pl.BlockSpec((1, fo), lambda r: (0, 0)),
        ]
    )
    out_specs = [pl.BlockSpec((_R, w), lambda r: (r, 0)) for w in out_widths]
    out_shape = [jax.ShapeDtypeStruct((N, w), jnp.float32) for w in out_widths]

    outs = pl.pallas_call(
        body,
        grid=(N // _R,),
        in_specs=in_specs,
        out_specs=out_specs,
        out_shape=out_shape,
    )(*h_blocks, *g_parts, ean, inv, w_self, w_nbr, w_edge, bias)
    return list(outs)


def _pad_axis(a, axis, to):
    pad = [(0, 0), (0, 0)]
    pad[axis] = (0, to - a.shape[axis])
    return jnp.pad(a, pad) if to > a.shape[axis] else a


def kernel(x, edge_index, edge_attr, params):
    src3d = edge_index[0].reshape(NW, CPT, K)
    dst3d = edge_index[1].reshape(NW, CPT, K)

    eid3d = jax.lax.iota(jnp.int32, E).reshape(NW, CPT, K)
    ea_cat = jnp.concatenate(
        [edge_attr,
         jnp.ones((E, 1), jnp.float32),
         jnp.zeros((E, 111), jnp.float32)], axis=1)
    pre = _SEG_SUM_128(ea_cat, eid3d, dst3d)
    ean, inv = _finalize_prepro(pre)

    h_blocks = [x]
    n_layers = len(params)
    for i, p in enumerate(params):
        fo = p['W_self'].shape[1]
        fi_pad = 128 * len(h_blocks)
        fo_pad = 128 if 1 < fo < 128 else fo
        w_self = _pad_axis(_pad_axis(p['W_self'], 0, fi_pad), 1, fo_pad)
        w_nbr = _pad_axis(_pad_axis(p['W_nbr'], 0, fi_pad), 1, fo_pad)
        w_edge = _pad_axis(p['W_edge'], 1, fo_pad)
        bias = _pad_axis(p['b'].reshape(1, fo), 1, fo_pad)
        out_widths = [1] if fo == 1 else [128] * (fo_pad // 128)

        g_parts = [_SEG_SUM_128(hb, src3d, dst3d) for hb in h_blocks]
        h_blocks = _dense_layer(
            h_blocks, g_parts, ean, inv, w_self, w_nbr, w_edge, bias,
            [128] * len(h_blocks), out_widths, relu=(i < n_layers - 1))

    return h_blocks[0]
